# baseline (speedup 1.0000x reference)
import dataclasses
import functools

import jax
import jax.numpy as jnp
from jax import lax
from jax.experimental import pallas as pl
from jax.experimental.pallas import tpu as pltpu
from jax.experimental.pallas import tpu_sc as plsc

N = 10000
NP = 10240
E = 320000
D = 128
H = 128
C = 10
G = 128

NBLK = 10
BLKN = NP // NBLK
ECH = 8000
NCH = E // ECH
FPT = 4

_PREC = jax.lax.Precision.HIGHEST


def _sc_compiler_params():
    cp = pltpu.CompilerParams()
    if "needs_layout_passes" in pltpu.CompilerParams.__dataclass_fields__:
        cp = dataclasses.replace(cp, needs_layout_passes=False)
    return cp


def _t1_body(w_ref, x_ref, out_ref):
    out_ref[...] = lax.dot_general(
        w_ref[...], x_ref[...], (((0,), (1,)), ((), ())),
        preferred_element_type=jnp.float32, precision=_PREC)


def _t1(x_pad, wcat1):
    return pl.pallas_call(
        _t1_body,
        grid=(NBLK,),
        in_specs=[
            pl.BlockSpec((D, 2 * H), lambda i: (0, 0)),
            pl.BlockSpec((BLKN, D), lambda i: (i, 0)),
        ],
        out_specs=pl.BlockSpec((2 * H, BLKN), lambda i: (0, i)),
        out_shape=jax.ShapeDtypeStruct((2 * H, NP), jnp.float32),
    )(wcat1, x_pad)


def _scatter_body(yz_hbm, pk_hbm, agg_hbm, t0, t1, t2, t3, a0, a1, a2, a3,
                  ebuf0, ebuf1, sem_t, sem_e0, sem_e1):
    wid = lax.axis_index("s") * 2 + lax.axis_index("c")
    row0 = wid * FPT
    tabs = (t0, t1, t2, t3)
    aggs = (a0, a1, a2, a3)

    tdmas = [pltpu.async_copy(yz_hbm.at[row0 + f], tabs[f], sem_t)
             for f in range(FPT)]

    @pl.loop(0, NP, step=64)
    def _zero(i):
        z = jnp.zeros((16,), jnp.float32)
        for f in range(FPT):
            for j in range(4):
                aggs[f][pl.ds(i + 16 * j, 16)] = z

    esems = (sem_e0, sem_e1)
    ebufs = (ebuf0, ebuf1)

    def _start(c, b):
        pltpu.async_copy(pk_hbm.at[pl.ds(c * ECH, ECH)], ebufs[b], esems[b])

    def _wait(b):
        pltpu.make_async_copy(pk_hbm.at[pl.ds(0, ECH)], ebufs[b],
                              esems[b]).wait()

    def _process(b):
        ebv = ebufs[b]

        @plsc.parallel_loop(0, ECH, 16, unroll=8)
        def _edges(g, ebv=ebv):
            vp = ebv[pl.ds(g, 16)]
            vsrc = lax.shift_right_logical(vp, 14)
            vdst = jnp.bitwise_and(vp, 16383)
            for f in range(FPT):
                vals = plsc.load_gather(tabs[f], [vsrc])
                plsc.addupdate_scatter(aggs[f], [vdst], vals)

    _start(0, 0)
    _start(1, 1)
    for d in tdmas:
        d.wait()

    @pl.loop(0, NCH // 2 - 1)
    def _pair(p):
        _wait(0)
        _process(0)
        _start(2 * p + 2, 0)
        _wait(1)
        _process(1)
        _start(2 * p + 3, 1)

    _wait(0)
    _process(0)
    _wait(1)
    _process(1)

    for f in range(FPT):
        pltpu.sync_copy(aggs[f], agg_hbm.at[row0 + f])


def _sc_scatter(yz_t, packed):
    mesh = plsc.VectorSubcoreMesh(core_axis_name="c", subcore_axis_name="s",
                                  num_cores=2, num_subcores=16)
    k = pl.kernel(
        _scatter_body,
        out_type=jax.ShapeDtypeStruct((H, NP), jnp.float32),
        mesh=mesh,
        scratch_types=[
            pltpu.VMEM((NP,), jnp.float32),
            pltpu.VMEM((NP,), jnp.float32),
            pltpu.VMEM((NP,), jnp.float32),
            pltpu.VMEM((NP,), jnp.float32),
            pltpu.VMEM((NP,), jnp.float32),
            pltpu.VMEM((NP,), jnp.float32),
            pltpu.VMEM((NP,), jnp.float32),
            pltpu.VMEM((NP,), jnp.float32),
            pltpu.VMEM((ECH,), jnp.int32),
            pltpu.VMEM((ECH,), jnp.int32),
            pltpu.SemaphoreType.DMA,
            pltpu.SemaphoreType.DMA,
            pltpu.SemaphoreType.DMA,
        ],
        compiler_params=_sc_compiler_params(),
    )
    return k(yz_t, packed)


def _t2_body(agg_ref, z_ref, b_ref, w_ref, out_ref):
    h = jnp.maximum(agg_ref[...] + z_ref[...] + b_ref[...], 0.0)
    out_ref[...] = lax.dot_general(
        w_ref[...], h, (((0,), (0,)), ((), ())),
        preferred_element_type=jnp.float32, precision=_PREC)


def _t2(agg_t, yz_t, b_col, wcat_next):
    return pl.pallas_call(
        _t2_body,
        grid=(NBLK,),
        in_specs=[
            pl.BlockSpec((H, BLKN), lambda i: (0, i)),
            pl.BlockSpec((H, BLKN), lambda i: (1, i)),
            pl.BlockSpec((H, 1), lambda i: (0, 0)),
            pl.BlockSpec((D, 2 * H), lambda i: (0, 0)),
        ],
        out_specs=pl.BlockSpec((2 * H, BLKN), lambda i: (0, i)),
        out_shape=jax.ShapeDtypeStruct((2 * H, NP), jnp.float32),
    )(agg_t, yz_t, b_col, wcat_next)


def _t3_body(agg_ref, z_ref, b_ref, batch_ref, wlin_ref, blin_ref, out_ref,
             acc_ref, cnt_ref):
    j = pl.program_id(0)
    h = agg_ref[...] + z_ref[...] + b_ref[...]
    bb = batch_ref[...]
    gids = lax.broadcasted_iota(jnp.int32, (BLKN, G), 1)
    oh = (bb == gids).astype(jnp.float32)

    @pl.when(j == 0)
    def _init():
        acc_ref[...] = jnp.zeros_like(acc_ref)
        cnt_ref[...] = jnp.zeros_like(cnt_ref)

    acc_ref[...] += lax.dot_general(
        h, oh, (((1,), (0,)), ((), ())),
        preferred_element_type=jnp.float32, precision=_PREC)
    cnt_ref[...] += jnp.sum(oh, axis=0, keepdims=True)

    @pl.when(j == NBLK - 1)
    def _fin():
        cnts = jnp.maximum(cnt_ref[...], 1.0)
        pooled_t = acc_ref[...] / cnts
        out = lax.dot_general(
            pooled_t, wlin_ref[...], (((0,), (1,)), ((), ())),
            preferred_element_type=jnp.float32, precision=_PREC)
        out_ref[...] = out + blin_ref[...]


def _t3(agg_t, yz_t, b_col, batch_col, w_lin, b_lin_row):
    return pl.pallas_call(
        _t3_body,
        grid=(NBLK,),
        in_specs=[
            pl.BlockSpec((H, BLKN), lambda i: (0, i)),
            pl.BlockSpec((H, BLKN), lambda i: (1, i)),
            pl.BlockSpec((H, 1), lambda i: (0, 0)),
            pl.BlockSpec((BLKN, 1), lambda i: (i, 0)),
            pl.BlockSpec((C, H), lambda i: (0, 0)),
            pl.BlockSpec((1, C), lambda i: (0, 0)),
        ],
        out_specs=pl.BlockSpec((G, C), lambda i: (0, 0)),
        out_shape=jax.ShapeDtypeStruct((G, C), jnp.float32),
        scratch_shapes=[
            pltpu.VMEM((H, G), jnp.float32),
            pltpu.VMEM((1, G), jnp.float32),
        ],
    )(agg_t, yz_t, b_col, batch_col, w_lin, b_lin_row)


def kernel(x, edge_index, batch, W_rel1, b_rel1, W_root1, W_rel2, b_rel2,
           W_root2, W_rel3, b_rel3, W_root3, W_lin, b_lin):
    x_pad = jnp.pad(x, ((0, NP - N), (0, 0)))
    batch_col = jnp.pad(batch, (0, NP - N),
                        constant_values=G).reshape(NP, 1)
    packed = jnp.bitwise_or(jnp.left_shift(edge_index[0], 14), edge_index[1])
    wcat1 = jnp.concatenate([W_rel1.T, W_root1.T], axis=1)
    wcat2 = jnp.concatenate([W_rel2.T, W_root2.T], axis=1)
    wcat3 = jnp.concatenate([W_rel3.T, W_root3.T], axis=1)
    b1 = b_rel1.reshape(H, 1)
    b2 = b_rel2.reshape(H, 1)
    b3 = b_rel3.reshape(H, 1)
    b_lin_row = b_lin.reshape(1, C)

    yz1 = _t1(x_pad, wcat1)
    agg1 = _sc_scatter(yz1, packed)
    yz2 = _t2(agg1, yz1, b1, wcat2)
    agg2 = _sc_scatter(yz2, packed)
    yz3 = _t2(agg2, yz2, b2, wcat3)
    agg3 = _sc_scatter(yz3, packed)
    out = _t3(agg3, yz3, b3, batch_col, W_lin, b_lin_row)
    return out


# speedup vs baseline: 1.1439x; 1.1439x over previous
import dataclasses
import functools

import jax
import jax.numpy as jnp
from jax import lax
from jax.experimental import pallas as pl
from jax.experimental.pallas import tpu as pltpu
from jax.experimental.pallas import tpu_sc as plsc

N = 10000
NP = 10240
E = 320000
D = 128
H = 128
C = 10
G = 128

NBLK = 10
BLKN = NP // NBLK
ECH = 8000
NCH = E // ECH
FPT = 4

_PREC = jax.lax.Precision.HIGHEST


def _sc_compiler_params():
    cp = pltpu.CompilerParams()
    if "needs_layout_passes" in pltpu.CompilerParams.__dataclass_fields__:
        cp = dataclasses.replace(cp, needs_layout_passes=False)
    return cp


def _pack_pairs(y):
    ye = lax.slice(y, (0, 0), (H // 2, y.shape[1]))
    yo = lax.slice(y, (H // 2, 0), (H, y.shape[1]))
    ue = lax.bitcast_convert_type(ye.astype(jnp.bfloat16),
                                  jnp.uint16).astype(jnp.uint32)
    uo = lax.bitcast_convert_type(yo.astype(jnp.bfloat16),
                                  jnp.uint16).astype(jnp.uint32)
    return lax.bitcast_convert_type((uo << 16) | ue, jnp.int32)


def _t1_body(w_ref, x_ref, out_ref, pk_ref):
    full = lax.dot_general(
        w_ref[...], x_ref[...], (((0,), (1,)), ((), ())),
        preferred_element_type=jnp.float32, precision=_PREC)
    out_ref[...] = full
    pk_ref[...] = _pack_pairs(full[:H, :])


def _t1(x_pad, wcat1):
    return pl.pallas_call(
        _t1_body,
        grid=(NBLK,),
        in_specs=[
            pl.BlockSpec((D, 2 * H), lambda i: (0, 0)),
            pl.BlockSpec((BLKN, D), lambda i: (i, 0)),
        ],
        out_specs=[
            pl.BlockSpec((2 * H, BLKN), lambda i: (0, i)),
            pl.BlockSpec((H // 2, BLKN), lambda i: (0, i)),
        ],
        out_shape=[
            jax.ShapeDtypeStruct((2 * H, NP), jnp.float32),
            jax.ShapeDtypeStruct((H // 2, NP), jnp.int32),
        ],
    )(wcat1, x_pad)


def _scatter_body(ypk_hbm, pk_hbm, agg_hbm, t0, t1, a0, a1, a2, a3,
                  ebuf0, ebuf1, sem_t, sem_e0, sem_e1):
    wid = lax.axis_index("s") * 2 + lax.axis_index("c")
    tabs = (t0, t1)
    aggs = (a0, a1, a2, a3)

    tdmas = [pltpu.async_copy(ypk_hbm.at[2 * wid + k], tabs[k], sem_t)
             for k in range(2)]

    @pl.loop(0, NP, step=64)
    def _zero(i):
        z = jnp.zeros((16,), jnp.float32)
        for f in range(FPT):
            for j in range(4):
                aggs[f][pl.ds(i + 16 * j, 16)] = z

    esems = (sem_e0, sem_e1)
    ebufs = (ebuf0, ebuf1)

    def _start(c, b):
        pltpu.async_copy(pk_hbm.at[pl.ds(c * ECH, ECH)], ebufs[b], esems[b])

    def _wait(b):
        pltpu.make_async_copy(pk_hbm.at[pl.ds(0, ECH)], ebufs[b],
                              esems[b]).wait()

    def _process(b):
        ebv = ebufs[b]

        @plsc.parallel_loop(0, ECH, 16, unroll=8)
        def _edges(g, ebv=ebv):
            vp = ebv[pl.ds(g, 16)]
            vsrc = lax.shift_right_logical(vp, 14)
            vdst = jnp.bitwise_and(vp, 16383)
            for k in range(2):
                vpk = plsc.load_gather(tabs[k], [vsrc])
                lo = plsc.bitcast(lax.shift_left(vpk, 16), jnp.float32)
                hi = plsc.bitcast(
                    jnp.bitwise_and(vpk, jnp.int32(-65536)), jnp.float32)
                plsc.addupdate_scatter(aggs[k], [vdst], lo)
                plsc.addupdate_scatter(aggs[2 + k], [vdst], hi)

    _start(0, 0)
    _start(1, 1)
    for d in tdmas:
        d.wait()

    @pl.loop(0, NCH // 2 - 1)
    def _pair(p):
        _wait(0)
        _process(0)
        _start(2 * p + 2, 0)
        _wait(1)
        _process(1)
        _start(2 * p + 3, 1)

    _wait(0)
    _process(0)
    _wait(1)
    _process(1)

    pltpu.sync_copy(aggs[0], agg_hbm.at[2 * wid])
    pltpu.sync_copy(aggs[1], agg_hbm.at[2 * wid + 1])
    pltpu.sync_copy(aggs[2], agg_hbm.at[H // 2 + 2 * wid])
    pltpu.sync_copy(aggs[3], agg_hbm.at[H // 2 + 2 * wid + 1])


def _sc_scatter(ypk, packed):
    mesh = plsc.VectorSubcoreMesh(core_axis_name="c", subcore_axis_name="s",
                                  num_cores=2, num_subcores=16)
    k = pl.kernel(
        _scatter_body,
        out_type=jax.ShapeDtypeStruct((H, NP), jnp.float32),
        mesh=mesh,
        scratch_types=[
            pltpu.VMEM((NP,), jnp.int32),
            pltpu.VMEM((NP,), jnp.int32),
            pltpu.VMEM((NP,), jnp.float32),
            pltpu.VMEM((NP,), jnp.float32),
            pltpu.VMEM((NP,), jnp.float32),
            pltpu.VMEM((NP,), jnp.float32),
            pltpu.VMEM((ECH,), jnp.int32),
            pltpu.VMEM((ECH,), jnp.int32),
            pltpu.SemaphoreType.DMA,
            pltpu.SemaphoreType.DMA,
            pltpu.SemaphoreType.DMA,
        ],
        compiler_params=_sc_compiler_params(),
    )
    return k(ypk, packed)


def _t2_body(agg_ref, z_ref, b_ref, w_ref, out_ref, pk_ref):
    h = jnp.maximum(agg_ref[...] + z_ref[...] + b_ref[...], 0.0)
    full = lax.dot_general(
        w_ref[...], h, (((0,), (0,)), ((), ())),
        preferred_element_type=jnp.float32, precision=_PREC)
    out_ref[...] = full
    pk_ref[...] = _pack_pairs(full[:H, :])


def _t2(agg_t, yz_t, b_col, wcat_next):
    return pl.pallas_call(
        _t2_body,
        grid=(NBLK,),
        in_specs=[
            pl.BlockSpec((H, BLKN), lambda i: (0, i)),
            pl.BlockSpec((H, BLKN), lambda i: (1, i)),
            pl.BlockSpec((H, 1), lambda i: (0, 0)),
            pl.BlockSpec((D, 2 * H), lambda i: (0, 0)),
        ],
        out_specs=[
            pl.BlockSpec((2 * H, BLKN), lambda i: (0, i)),
            pl.BlockSpec((H // 2, BLKN), lambda i: (0, i)),
        ],
        out_shape=[
            jax.ShapeDtypeStruct((2 * H, NP), jnp.float32),
            jax.ShapeDtypeStruct((H // 2, NP), jnp.int32),
        ],
    )(agg_t, yz_t, b_col, wcat_next)


def _t3_body(agg_ref, z_ref, b_ref, batch_ref, wlin_ref, blin_ref, out_ref,
             acc_ref, cnt_ref):
    j = pl.program_id(0)
    h = agg_ref[...] + z_ref[...] + b_ref[...]
    bb = batch_ref[...]
    gids = lax.broadcasted_iota(jnp.int32, (BLKN, G), 1)
    oh = (bb == gids).astype(jnp.float32)

    @pl.when(j == 0)
    def _init():
        acc_ref[...] = jnp.zeros_like(acc_ref)
        cnt_ref[...] = jnp.zeros_like(cnt_ref)

    acc_ref[...] += lax.dot_general(
        h, oh, (((1,), (0,)), ((), ())),
        preferred_element_type=jnp.float32, precision=_PREC)
    cnt_ref[...] += jnp.sum(oh, axis=0, keepdims=True)

    @pl.when(j == NBLK - 1)
    def _fin():
        cnts = jnp.maximum(cnt_ref[...], 1.0)
        pooled_t = acc_ref[...] / cnts
        out = lax.dot_general(
            pooled_t, wlin_ref[...], (((0,), (1,)), ((), ())),
            preferred_element_type=jnp.float32, precision=_PREC)
        out_ref[...] = out + blin_ref[...]


def _t3(agg_t, yz_t, b_col, batch_col, w_lin, b_lin_row):
    return pl.pallas_call(
        _t3_body,
        grid=(NBLK,),
        in_specs=[
            pl.BlockSpec((H, BLKN), lambda i: (0, i)),
            pl.BlockSpec((H, BLKN), lambda i: (1, i)),
            pl.BlockSpec((H, 1), lambda i: (0, 0)),
            pl.BlockSpec((BLKN, 1), lambda i: (i, 0)),
            pl.BlockSpec((C, H), lambda i: (0, 0)),
            pl.BlockSpec((1, C), lambda i: (0, 0)),
        ],
        out_specs=pl.BlockSpec((G, C), lambda i: (0, 0)),
        out_shape=jax.ShapeDtypeStruct((G, C), jnp.float32),
        scratch_shapes=[
            pltpu.VMEM((H, G), jnp.float32),
            pltpu.VMEM((1, G), jnp.float32),
        ],
    )(agg_t, yz_t, b_col, batch_col, w_lin, b_lin_row)


def kernel(x, edge_index, batch, W_rel1, b_rel1, W_root1, W_rel2, b_rel2,
           W_root2, W_rel3, b_rel3, W_root3, W_lin, b_lin):
    x_pad = jnp.pad(x, ((0, NP - N), (0, 0)))
    batch_col = jnp.pad(batch, (0, NP - N),
                        constant_values=G).reshape(NP, 1)
    packed = jnp.bitwise_or(jnp.left_shift(edge_index[0], 14), edge_index[1])
    wcat1 = jnp.concatenate([W_rel1.T, W_root1.T], axis=1)
    wcat2 = jnp.concatenate([W_rel2.T, W_root2.T], axis=1)
    wcat3 = jnp.concatenate([W_rel3.T, W_root3.T], axis=1)
    b1 = b_rel1.reshape(H, 1)
    b2 = b_rel2.reshape(H, 1)
    b3 = b_rel3.reshape(H, 1)
    b_lin_row = b_lin.reshape(1, C)

    yz1, ypk1 = _t1(x_pad, wcat1)
    agg1 = _sc_scatter(ypk1, packed)
    yz2, ypk2 = _t2(agg1, yz1, b1, wcat2)
    agg2 = _sc_scatter(ypk2, packed)
    yz3, ypk3 = _t2(agg2, yz2, b2, wcat3)
    agg3 = _sc_scatter(ypk3, packed)
    out = _t3(agg3, yz3, b3, batch_col, W_lin, b_lin_row)
    return out
